# baseline (speedup 1.0000x reference)
import functools
import jax
from jax import lax
from jax.experimental import pallas as pl
from jax.experimental.pallas import tpu as pltpu
from jax.experimental.pallas import tpu_sc as plsc

_NC, _NS = 2, 16
_NW = _NC * _NS
_CHUNK = 64


@functools.lru_cache(maxsize=None)
def _make_sc_broadcast(B, L, D, dtype):
    rpw = L // _NW
    nchunk = rpw // _CHUNK
    mesh = plsc.VectorSubcoreMesh(core_axis_name="c", subcore_axis_name="s")

    @functools.partial(
        pl.kernel,
        mesh=mesh,
        out_type=jax.ShapeDtypeStruct((B, L, D), dtype),
        scratch_types=[pltpu.VMEM((_CHUNK, D), dtype)],
    )
    def sc_broadcast(table_hbm, out_hbm, buf):
        wid = lax.axis_index("s") * _NC + lax.axis_index("c")
        for c in range(nchunk):
            base = wid * rpw + c * _CHUNK
            pltpu.sync_copy(table_hbm.at[pl.ds(base, _CHUNK)], buf)
            for b in range(B):
                pltpu.sync_copy(buf, out_hbm.at[b, pl.ds(base, _CHUNK)])

    return sc_broadcast


def kernel(x, table):
    B, L, D = x.shape
    return _make_sc_broadcast(B, L, D, table.dtype)(table)
